# baseline (speedup 1.0000x reference)
import jax
import jax.numpy as jnp
from jax import lax
from jax.experimental import pallas as pl

N_TOK = 32768
D_MODEL = 768
N_EXP = 64
_BT = 8192


def _gate_body(x_ref, w_ref, idx_ref, gate_ref):
    logits = lax.dot_general(
        w_ref[...], x_ref[...],
        (((1,), (1,)), ((), ())),
        preferred_element_type=jnp.float32,
    )
    m = jnp.max(logits, axis=0, keepdims=True)
    ii = lax.broadcasted_iota(jnp.int32, (N_EXP, _BT), 0)
    cand = jnp.where(logits == m, ii, N_EXP)
    idx = jnp.min(cand, axis=0, keepdims=True)
    s = jnp.sum(jnp.exp(logits - m), axis=0, keepdims=True)
    idx_ref[...] = idx
    gate_ref[...] = 1.0 / s


def kernel(x, W):
    idx2, gate2 = pl.pallas_call(
        _gate_body,
        grid=(N_TOK // _BT,),
        in_specs=[
            pl.BlockSpec((_BT, D_MODEL), lambda i: (i, 0)),
            pl.BlockSpec((N_EXP, D_MODEL), lambda i: (0, 0)),
        ],
        out_specs=[
            pl.BlockSpec((1, _BT), lambda i: (0, i)),
            pl.BlockSpec((1, _BT), lambda i: (0, i)),
        ],
        out_shape=[
            jax.ShapeDtypeStruct((1, N_TOK), jnp.int32),
            jax.ShapeDtypeStruct((1, N_TOK), jnp.float32),
        ],
    )(x, W)
    expert_indices = idx2.reshape(N_TOK)
    expert_gates = gate2.reshape(N_TOK)
    load_balance_loss = jnp.zeros((), jnp.float32)
    return (expert_indices, expert_gates, load_balance_loss)


# speedup vs baseline: 1.0345x; 1.0345x over previous
import jax
import jax.numpy as jnp
from jax import lax
from jax.experimental import pallas as pl
from jax.experimental.pallas import tpu as pltpu

N_TOK = 32768
D_MODEL = 768
N_EXP = 64

_C = 2048
_NBUF = 4
_NSTEP = N_TOK // _C
_LEAD = _NBUF - 1


def _gate_body(x_hbm, w_ref, idx_ref, gate_ref, bufs, sems):
    def start(s):
        b = s % _NBUF
        pltpu.make_async_copy(
            x_hbm.at[pl.ds(s * _C, _C), :], bufs.at[b], sems.at[b]
        ).start()

    def compute(s):
        b = s % _NBUF
        pltpu.make_async_copy(
            x_hbm.at[pl.ds(s * _C, _C), :], bufs.at[b], sems.at[b]
        ).wait()
        logits = lax.dot_general(
            w_ref[...], bufs[b],
            (((1,), (1,)), ((), ())),
            preferred_element_type=jnp.float32,
        )
        m = jnp.max(logits, axis=0, keepdims=True)
        ii = lax.broadcasted_iota(jnp.int32, (N_EXP, _C), 0)
        cand = jnp.where(logits == m, ii, N_EXP)
        idx = jnp.min(cand, axis=0, keepdims=True)
        s_ = jnp.sum(jnp.exp(logits - m), axis=0, keepdims=True)
        idx_ref[:, pl.ds(s * _C, _C)] = idx
        gate_ref[:, pl.ds(s * _C, _C)] = 1.0 / s_

    for s in range(_LEAD):
        start(s)
    for s in range(_NSTEP):
        if s + _LEAD < _NSTEP:
            start(s + _LEAD)
        compute(s)


def kernel(x, W):
    idx2, gate2 = pl.pallas_call(
        _gate_body,
        in_specs=[
            pl.BlockSpec(memory_space=pl.ANY),
            pl.BlockSpec((N_EXP, D_MODEL), lambda: (0, 0)),
        ],
        out_specs=[
            pl.BlockSpec((1, N_TOK), lambda: (0, 0)),
            pl.BlockSpec((1, N_TOK), lambda: (0, 0)),
        ],
        out_shape=[
            jax.ShapeDtypeStruct((1, N_TOK), jnp.int32),
            jax.ShapeDtypeStruct((1, N_TOK), jnp.float32),
        ],
        scratch_shapes=[
            pltpu.VMEM((_NBUF, _C, D_MODEL), jnp.float32),
            pltpu.SemaphoreType.DMA((_NBUF,)),
        ],
    )(x, W)
    expert_indices = idx2.reshape(N_TOK)
    expert_gates = gate2.reshape(N_TOK)
    load_balance_loss = jnp.zeros((), jnp.float32)
    return (expert_indices, expert_gates, load_balance_loss)
